# baseline (speedup 1.0000x reference)
import functools

import jax
import jax.numpy as jnp
from jax import lax
from jax.experimental import pallas as pl
from jax.experimental.pallas import tpu as pltpu
from jax.experimental.pallas import tpu_sc as plsc

N = 10000
E = 320000
DIN = 128
DH = 256
DOUT = 128

NC = 2
NS = 16
CHUNK = 128

CPT1 = 160
CPT2 = 80
NROWS2D = NS * CPT1
E_PAD = NROWS2D * CHUNK

N_PAD = 10240
ROWS_TILE = N_PAD // NS
ROWS_IO = 128
IO_CHUNKS = ROWS_TILE // ROWS_IO
TRASH = N

HCH = 64

_mesh = plsc.VectorSubcoreMesh(core_axis_name="c", subcore_axis_name="s")


def _make_conv_scatter(dh, feature_split):
    cpt = CPT1 if feature_split else CPT2

    @functools.partial(
        pl.kernel,
        mesh=_mesh,
        out_type=jax.ShapeDtypeStruct((2, N_PAD, dh), jnp.float32),
        scratch_types=[
            pltpu.VMEM((CHUNK,), jnp.int32),
            pltpu.VMEM((CHUNK,), jnp.int32),
            pltpu.VMEM((CHUNK,), jnp.int32),
            pltpu.VMEM((CHUNK,), jnp.int32),
            pltpu.VMEM((CHUNK, dh), jnp.float32),
            pltpu.VMEM((CHUNK, dh), jnp.float32),
            pltpu.VMEM((ROWS_IO, dh), jnp.float32),
            pltpu.VMEM_SHARED((N_PAD, dh), jnp.float32),
            pltpu.SemaphoreType.DMA,
            pltpu.SemaphoreType.DMA,
        ],
    )
    def conv_kernel(in_hbm, src1_hbm, dst1_hbm, out_hbm,
                    sidx0_v, sidx1_v, didx0_v, didx1_v,
                    rows0_v, rows1_v, bounce_v, acc_sh,
                    sem0, sem1):
        c = lax.axis_index("c")
        s = lax.axis_index("s")
        rows = (rows0_v, rows1_v)
        sems = (sem0, sem1)
        if feature_split:
            gat_hbm = in_hbm.at[c]
            cbase = s * CPT1
        else:
            gat_hbm = in_hbm.at[0]
            cbase = (s * NC + c) * CPT2

        for i in range(IO_CHUNKS):
            rb = s * ROWS_TILE + i * ROWS_IO
            pltpu.sync_copy(in_hbm.at[c, pl.ds(rb, ROWS_IO)], bounce_v)
            pltpu.sync_copy(bounce_v, acc_sh.at[pl.ds(rb, ROWS_IO)])
        plsc.subcore_barrier()

        ebase = cbase * CHUNK

        def body(j, carry):
            off = ebase + j * CHUNK
            pltpu.sync_copy(src1_hbm.at[pl.ds(off, CHUNK)], sidx0_v)
            pltpu.sync_copy(dst1_hbm.at[pl.ds(off, CHUNK)], didx0_v)
            pltpu.async_copy(gat_hbm.at[sidx0_v], rows0_v, sem0).wait()
            pltpu.sync_copy(rows0_v, acc_sh.at[didx0_v], add=True)
            return carry

        lax.fori_loop(0, cpt, body, 0)
        plsc.subcore_barrier()
        for i in range(IO_CHUNKS):
            rb = s * ROWS_TILE + i * ROWS_IO
            pltpu.sync_copy(acc_sh.at[pl.ds(rb, ROWS_IO)], bounce_v)
            pltpu.sync_copy(bounce_v, out_hbm.at[c, pl.ds(rb, ROWS_IO)])

    return conv_kernel


_conv_scatter_1 = _make_conv_scatter(DH // 2, True)
_conv_scatter_2 = _make_conv_scatter(DOUT, False)


N_HCH = CPT2 * 2


@functools.partial(
    pl.kernel,
    mesh=_mesh,
    out_type=jax.ShapeDtypeStruct((E_PAD, 16), jnp.float32),
    scratch_types=[
        pltpu.VMEM((CPT2, CHUNK), jnp.int32),
        pltpu.VMEM((CPT2, CHUNK), jnp.int32),
        pltpu.VMEM((HCH, 2 * DOUT), jnp.float32),
        pltpu.VMEM((HCH, 2 * DOUT), jnp.float32),
        pltpu.VMEM((HCH, 2 * DOUT), jnp.float32),
        pltpu.VMEM((HCH, 2 * DOUT), jnp.float32),
        pltpu.VMEM((16, 16), jnp.float32),
        pltpu.VMEM((HCH, 16), jnp.float32),
        pltpu.VMEM((HCH, 16), jnp.float32),
        pltpu.SemaphoreType.DMA,
        pltpu.SemaphoreType.DMA,
        pltpu.SemaphoreType.DMA,
        pltpu.SemaphoreType.DMA,
        pltpu.SemaphoreType.DMA,
        pltpu.SemaphoreType.DMA,
    ],
)
def _dec_kernel(a_hbm, b_hbm, src2d_hbm, dst2d_hbm, w_hbm, p_hbm,
                sidx_v, didx_v, a0_v, a1_v, b0_v, b1_v, w_v, p0_v, p1_v,
                asem0, asem1, bsem0, bsem1, psem0, psem1):
    c = lax.axis_index("c")
    s = lax.axis_index("s")
    wid = s * NC + c
    cbase = wid * CPT2
    ebase = wid * (CPT2 * CHUNK)

    av = (a0_v, a1_v)
    bv = (b0_v, b1_v)
    pv = (p0_v, p1_v)
    asems = (asem0, asem1)
    bsems = (bsem0, bsem1)
    psems = (psem0, psem1)

    pltpu.sync_copy(w_hbm, w_v)
    wregs = [w_v[i, :] for i in range(16)]
    zero = jnp.zeros((16,), jnp.float32)

    pltpu.sync_copy(src2d_hbm.at[pl.ds(cbase, CPT2)], sidx_v)
    pltpu.sync_copy(dst2d_hbm.at[pl.ds(cbase, CPT2)], didx_v)

    def sidx_of(j):
        return sidx_v.at[j // 2, pl.ds((j % 2) * HCH, HCH)]

    def didx_of(j):
        return didx_v.at[j // 2, pl.ds((j % 2) * HCH, HCH)]

    for k in range(2):
        pltpu.async_copy(a_hbm.at[sidx_of(k)], av[k], asems[k])
        pltpu.async_copy(b_hbm.at[didx_of(k)], bv[k], bsems[k])

    def body(i, carry):
        for k in range(2):
            j = i * 2 + k
            pltpu.make_async_copy(a_hbm.at[sidx_of(j)], av[k], asems[k]).wait()
            pltpu.make_async_copy(b_hbm.at[didx_of(j)], bv[k], bsems[k]).wait()

            @pl.when(j >= 2)
            def _():
                pltpu.make_async_copy(
                    pv[k], p_hbm.at[pl.ds(ebase + j * HCH, HCH)],
                    psems[k]).wait()

            def row(r, rc):
                acc = [zero, zero, zero, zero]
                for jj in range(16):
                    t = jnp.maximum(
                        av[k][r, pl.ds(jj * 16, 16)]
                        + bv[k][r, pl.ds(jj * 16, 16)], 0.0)
                    acc[jj % 4] = t * wregs[jj] + acc[jj % 4]
                pv[k][r, :] = (acc[0] + acc[1]) + (acc[2] + acc[3])
                return rc

            lax.fori_loop(0, HCH, row, 0)
            pltpu.async_copy(pv[k], p_hbm.at[pl.ds(ebase + j * HCH, HCH)],
                             psems[k])
            jn = j + 2

            @pl.when(jn < N_HCH)
            def _():
                pltpu.async_copy(a_hbm.at[sidx_of(jn)], av[k], asems[k])
                pltpu.async_copy(b_hbm.at[didx_of(jn)], bv[k], bsems[k])
        return carry

    lax.fori_loop(0, N_HCH // 2, body, 0)
    for k in range(2):
        pltpu.make_async_copy(
            pv[k], p_hbm.at[pl.ds(ebase + (N_HCH - 2 + k) * HCH, HCH)],
            psems[k]).wait()


BM = 2048
BE = 2000
NHI = N_PAD // 128


def _deg_body(dst_ref, o_ref):
    i = pl.program_id(0)

    @pl.when(i == 0)
    def _():
        o_ref[...] = jnp.full((NHI, 128), 1.0, jnp.float32)

    d = dst_ref[...]
    hi = lax.broadcasted_iota(jnp.int32, (BE, NHI), 1)
    lo = lax.broadcasted_iota(jnp.int32, (BE, 128), 1)
    u = (d // 128 == hi).astype(jnp.float32)
    v = (d % 128 == lo).astype(jnp.float32)
    o_ref[...] += lax.dot_general(u, v, (((0,), (0,)), ((), ())),
                                  preferred_element_type=jnp.float32,
                 precision=lax.Precision.HIGHEST)


def _mm1_body(x_ref, w_ref, deg_ref, o_ref):
    dis = lax.rsqrt(deg_ref[...])
    h = jnp.dot(x_ref[...], w_ref[...], preferred_element_type=jnp.float32,
                 precision=lax.Precision.HIGHEST)
    o_ref[0] = dis * h[:, : DH // 2]
    o_ref[1] = dis * h[:, DH // 2:]


def _mm2_body(s1_ref, deg_ref, b1_ref, w2_ref, o_ref):
    dis = lax.rsqrt(deg_ref[...])
    b1 = b1_ref[...]
    w2 = w2_ref[...]
    zL = jnp.maximum(dis * s1_ref[0] + b1[:, : DH // 2], 0.0)
    zR = jnp.maximum(dis * s1_ref[1] + b1[:, DH // 2:], 0.0)
    o_ref[0] = dis * (
        jnp.dot(zL, w2[: DH // 2], preferred_element_type=jnp.float32,
                 precision=lax.Precision.HIGHEST)
        + jnp.dot(zR, w2[DH // 2:], preferred_element_type=jnp.float32,
                 precision=lax.Precision.HIGHEST))
    o_ref[1] = jnp.zeros((BM, DOUT), jnp.float32)


def _mm3_body(s2_ref, deg_ref, b2_ref, wc_ref, bc_ref,
              oA_ref, oB_ref):
    dis = lax.rsqrt(deg_ref[...])
    z = dis * (s2_ref[0] + s2_ref[1]) + b2_ref[...]
    ab = jnp.dot(z, wc_ref[...], preferred_element_type=jnp.float32,
                 precision=lax.Precision.HIGHEST) \
        + bc_ref[...]
    oA_ref[...] = ab[:, :DH]
    oB_ref[...] = ab[:, DH:]


BR = 3200


def _red_body(p_ref, b_ref, o_ref):
    o_ref[...] = jnp.sum(p_ref[...], axis=1, keepdims=True) + b_ref[...]


def kernel(x, edge_index, W1, b1, W2, b2, lpW1, lpb1, lpW2, lpb2):
    src = edge_index[0]
    dst = edge_index[1]

    pad = E_PAD - E
    src1 = jnp.concatenate([src, jnp.zeros((pad,), jnp.int32)])
    dst1 = jnp.concatenate([dst, jnp.full((pad,), TRASH, jnp.int32)])
    src2d = src1.reshape(NROWS2D, CHUNK)
    dst2d = dst1.reshape(NROWS2D, CHUNK)
    xp = jnp.pad(x, ((0, N_PAD - N), (0, 0)))

    deg_mat = pl.pallas_call(
        _deg_body,
        grid=(E // BE,),
        in_specs=[pl.BlockSpec((BE, 1), lambda i: (i, 0))],
        out_specs=pl.BlockSpec((NHI, 128), lambda i: (0, 0)),
        out_shape=jax.ShapeDtypeStruct((NHI, 128), jnp.float32),
    )(dst.reshape(E, 1))
    deg_col = deg_mat.reshape(N_PAD, 1)

    h1p = pl.pallas_call(
        _mm1_body,
        grid=(N_PAD // BM,),
        in_specs=[
            pl.BlockSpec((BM, DIN), lambda i: (i, 0)),
            pl.BlockSpec((DIN, DH), lambda i: (0, 0)),
            pl.BlockSpec((BM, 1), lambda i: (i, 0)),
        ],
        out_specs=pl.BlockSpec((2, BM, DH // 2), lambda i: (0, i, 0)),
        out_shape=jax.ShapeDtypeStruct((2, N_PAD, DH // 2), jnp.float32),
    )(xp, W1, deg_col)

    s1 = _conv_scatter_1(h1p, src1, dst1)

    h2p = pl.pallas_call(
        _mm2_body,
        grid=(N_PAD // BM,),
        in_specs=[
            pl.BlockSpec((2, BM, DH // 2), lambda i: (0, i, 0)),
            pl.BlockSpec((BM, 1), lambda i: (i, 0)),
            pl.BlockSpec((1, DH), lambda i: (0, 0)),
            pl.BlockSpec((DH, DOUT), lambda i: (0, 0)),
        ],
        out_specs=pl.BlockSpec((2, BM, DOUT), lambda i: (0, i, 0)),
        out_shape=jax.ShapeDtypeStruct((2, N_PAD, DOUT), jnp.float32),
    )(s1, deg_col, b1.reshape(1, DH), W2)

    s2 = _conv_scatter_2(h2p, src1, dst1)

    wcat = jnp.concatenate([lpW1[:DOUT], lpW1[DOUT:]], axis=1)
    bcat = jnp.concatenate([lpb1, jnp.zeros((DH,), jnp.float32)]).reshape(1, -1)

    A, B = pl.pallas_call(
        _mm3_body,
        grid=(N_PAD // BM,),
        in_specs=[
            pl.BlockSpec((2, BM, DOUT), lambda i: (0, i, 0)),
            pl.BlockSpec((BM, 1), lambda i: (i, 0)),
            pl.BlockSpec((1, DOUT), lambda i: (0, 0)),
            pl.BlockSpec((DOUT, 2 * DH), lambda i: (0, 0)),
            pl.BlockSpec((1, 2 * DH), lambda i: (0, 0)),
        ],
        out_specs=[pl.BlockSpec((BM, DH), lambda i: (i, 0))] * 2,
        out_shape=[jax.ShapeDtypeStruct((N_PAD, DH), jnp.float32)] * 2,
    )(s2, deg_col, b2.reshape(1, DOUT), wcat, bcat)

    w16 = lpW2.reshape(16, 16)

    P = _dec_kernel(A, B, src2d, dst2d, w16)

    out = pl.pallas_call(
        _red_body,
        grid=(E // BR,),
        in_specs=[
            pl.BlockSpec((BR, 16), lambda i: (i, 0)),
            pl.BlockSpec((1, 1), lambda i: (0, 0)),
        ],
        out_specs=pl.BlockSpec((BR, 1), lambda i: (i, 0)),
        out_shape=jax.ShapeDtypeStruct((E, 1), jnp.float32),
    )(P, lpb2.reshape(1, 1))

    return out


# speedup vs baseline: 1.0061x; 1.0061x over previous
import functools

import jax
import jax.numpy as jnp
from jax import lax
from jax.experimental import pallas as pl
from jax.experimental.pallas import tpu as pltpu
from jax.experimental.pallas import tpu_sc as plsc

N = 10000
E = 320000
DIN = 128
DH = 256
DOUT = 128

NC = 2
NS = 16
CHUNK = 128

CPT1 = 160
CPT2 = 80
NROWS2D = NS * CPT1
E_PAD = NROWS2D * CHUNK

N_PAD = 10240
ROWS_TILE = N_PAD // NS
ROWS_IO = 128
IO_CHUNKS = ROWS_TILE // ROWS_IO
TRASH = N

HCH = 64

_mesh = plsc.VectorSubcoreMesh(core_axis_name="c", subcore_axis_name="s")


def _make_conv_scatter(dh, feature_split):
    cpt = CPT1 if feature_split else CPT2

    @functools.partial(
        pl.kernel,
        mesh=_mesh,
        out_type=[jax.ShapeDtypeStruct((N_PAD, dh), jnp.float32)] * 2,
        scratch_types=[
            pltpu.VMEM((CHUNK,), jnp.int32),
            pltpu.VMEM((CHUNK,), jnp.int32),
            pltpu.VMEM((CHUNK, dh), jnp.float32),
            pltpu.VMEM((ROWS_IO, dh), jnp.float32),
            pltpu.VMEM_SHARED((N_PAD, dh), jnp.float32),
            pltpu.SemaphoreType.DMA,
        ],
    )
    def conv_kernel(inL_hbm, inR_hbm, src_hbm, dst_hbm, outL_hbm, outR_hbm,
                    sidx_v, didx_v, rows_v, bounce_v, acc_sh, sem):
        c = lax.axis_index("c")
        s = lax.axis_index("s")

        def run(init_hbm, gat_hbm, out_hbm, cbase):
            for i in range(IO_CHUNKS):
                rb = s * ROWS_TILE + i * ROWS_IO
                pltpu.sync_copy(init_hbm.at[pl.ds(rb, ROWS_IO)], bounce_v)
                pltpu.sync_copy(bounce_v, acc_sh.at[pl.ds(rb, ROWS_IO)])
            plsc.subcore_barrier()

            ebase = cbase * CHUNK

            def body(j, carry):
                off = ebase + j * CHUNK
                pltpu.sync_copy(src_hbm.at[pl.ds(off, CHUNK)], sidx_v)
                pltpu.sync_copy(dst_hbm.at[pl.ds(off, CHUNK)], didx_v)
                pltpu.async_copy(gat_hbm.at[sidx_v], rows_v, sem).wait()
                pltpu.sync_copy(rows_v, acc_sh.at[didx_v], add=True)
                return carry

            lax.fori_loop(0, cpt, body, 0)
            plsc.subcore_barrier()
            for i in range(IO_CHUNKS):
                rb = s * ROWS_TILE + i * ROWS_IO
                pltpu.sync_copy(acc_sh.at[pl.ds(rb, ROWS_IO)], bounce_v)
                pltpu.sync_copy(bounce_v, out_hbm.at[pl.ds(rb, ROWS_IO)])

        if feature_split:
            @pl.when(c == 0)
            def _():
                run(inL_hbm, inL_hbm, outL_hbm, s * CPT1)

            @pl.when(c == 1)
            def _():
                run(inR_hbm, inR_hbm, outR_hbm, s * CPT1)
        else:
            wid = s * NC + c

            @pl.when(c == 0)
            def _():
                run(inL_hbm, inL_hbm, outL_hbm, wid * CPT2)

            @pl.when(c == 1)
            def _():
                run(inR_hbm, inL_hbm, outR_hbm, wid * CPT2)

    return conv_kernel


_conv_scatter_1 = _make_conv_scatter(DH // 2, True)
_conv_scatter_2 = _make_conv_scatter(DOUT, False)


N_HCH = CPT2 * 2


@functools.partial(
    pl.kernel,
    mesh=_mesh,
    out_type=jax.ShapeDtypeStruct((E_PAD, 16), jnp.float32),
    scratch_types=[
        pltpu.VMEM((CPT2, CHUNK), jnp.int32),
        pltpu.VMEM((CPT2, CHUNK), jnp.int32),
        pltpu.VMEM((HCH, 2 * DOUT), jnp.float32),
        pltpu.VMEM((HCH, 2 * DOUT), jnp.float32),
        pltpu.VMEM((HCH, 2 * DOUT), jnp.float32),
        pltpu.VMEM((HCH, 2 * DOUT), jnp.float32),
        pltpu.VMEM((16, 16), jnp.float32),
        pltpu.VMEM((HCH, 16), jnp.float32),
        pltpu.VMEM((HCH, 16), jnp.float32),
        pltpu.SemaphoreType.DMA,
        pltpu.SemaphoreType.DMA,
        pltpu.SemaphoreType.DMA,
        pltpu.SemaphoreType.DMA,
        pltpu.SemaphoreType.DMA,
        pltpu.SemaphoreType.DMA,
    ],
)
def _dec_kernel(a_hbm, b_hbm, src2d_hbm, dst2d_hbm, w_hbm, p_hbm,
                sidx_v, didx_v, a0_v, a1_v, b0_v, b1_v, w_v, p0_v, p1_v,
                asem0, asem1, bsem0, bsem1, psem0, psem1):
    c = lax.axis_index("c")
    s = lax.axis_index("s")
    wid = s * NC + c
    cbase = wid * CPT2
    ebase = wid * (CPT2 * CHUNK)

    av = (a0_v, a1_v)
    bv = (b0_v, b1_v)
    pv = (p0_v, p1_v)
    asems = (asem0, asem1)
    bsems = (bsem0, bsem1)
    psems = (psem0, psem1)

    pltpu.sync_copy(w_hbm, w_v)
    wregs = [w_v[i, :] for i in range(16)]
    zero = jnp.zeros((16,), jnp.float32)

    pltpu.sync_copy(src2d_hbm.at[pl.ds(cbase, CPT2)], sidx_v)
    pltpu.sync_copy(dst2d_hbm.at[pl.ds(cbase, CPT2)], didx_v)

    def sidx_of(j):
        return sidx_v.at[j // 2, pl.ds((j % 2) * HCH, HCH)]

    def didx_of(j):
        return didx_v.at[j // 2, pl.ds((j % 2) * HCH, HCH)]

    for k in range(2):
        pltpu.async_copy(a_hbm.at[sidx_of(k)], av[k], asems[k])
        pltpu.async_copy(b_hbm.at[didx_of(k)], bv[k], bsems[k])

    def body(i, carry):
        for k in range(2):
            j = i * 2 + k
            pltpu.make_async_copy(a_hbm.at[sidx_of(j)], av[k], asems[k]).wait()
            pltpu.make_async_copy(b_hbm.at[didx_of(j)], bv[k], bsems[k]).wait()

            @pl.when(j >= 2)
            def _():
                pltpu.make_async_copy(
                    pv[k], p_hbm.at[pl.ds(ebase + j * HCH, HCH)],
                    psems[k]).wait()

            def row(r, rc):
                acc = [zero, zero, zero, zero]
                for jj in range(16):
                    t = jnp.maximum(
                        av[k][r, pl.ds(jj * 16, 16)]
                        + bv[k][r, pl.ds(jj * 16, 16)], 0.0)
                    acc[jj % 4] = t * wregs[jj] + acc[jj % 4]
                pv[k][r, :] = (acc[0] + acc[1]) + (acc[2] + acc[3])
                return rc

            lax.fori_loop(0, HCH, row, 0)
            pltpu.async_copy(pv[k], p_hbm.at[pl.ds(ebase + j * HCH, HCH)],
                             psems[k])
            jn = j + 2

            @pl.when(jn < N_HCH)
            def _():
                pltpu.async_copy(a_hbm.at[sidx_of(jn)], av[k], asems[k])
                pltpu.async_copy(b_hbm.at[didx_of(jn)], bv[k], bsems[k])
        return carry

    lax.fori_loop(0, N_HCH // 2, body, 0)
    for k in range(2):
        pltpu.make_async_copy(
            pv[k], p_hbm.at[pl.ds(ebase + (N_HCH - 2 + k) * HCH, HCH)],
            psems[k]).wait()


BM = 2048
BE = 2000
NHI = N_PAD // 128


def _deg_body(dst_ref, o_ref):
    i = pl.program_id(0)

    @pl.when(i == 0)
    def _():
        o_ref[...] = jnp.full((NHI, 128), 1.0, jnp.float32)

    d = dst_ref[...]
    hi = lax.broadcasted_iota(jnp.int32, (BE, NHI), 1)
    lo = lax.broadcasted_iota(jnp.int32, (BE, 128), 1)
    u = (d // 128 == hi).astype(jnp.float32)
    v = (d % 128 == lo).astype(jnp.float32)
    o_ref[...] += lax.dot_general(u, v, (((0,), (0,)), ((), ())),
                                  preferred_element_type=jnp.float32,
                 precision=lax.Precision.HIGHEST)


def _mm1_body(x_ref, w_ref, deg_ref, oL_ref, oR_ref):
    dis = lax.rsqrt(deg_ref[...])
    h = jnp.dot(x_ref[...], w_ref[...], preferred_element_type=jnp.float32,
                 precision=lax.Precision.HIGHEST)
    oL_ref[...] = dis * h[:, : DH // 2]
    oR_ref[...] = dis * h[:, DH // 2:]


def _mm2_body(sL_ref, sR_ref, deg_ref, b1_ref, w2_ref, o_ref):
    dis = lax.rsqrt(deg_ref[...])
    b1 = b1_ref[...]
    w2 = w2_ref[...]
    zL = jnp.maximum(dis * sL_ref[...] + b1[:, : DH // 2], 0.0)
    zR = jnp.maximum(dis * sR_ref[...] + b1[:, DH // 2:], 0.0)
    o_ref[...] = dis * (
        jnp.dot(zL, w2[: DH // 2], preferred_element_type=jnp.float32,
                 precision=lax.Precision.HIGHEST)
        + jnp.dot(zR, w2[DH // 2:], preferred_element_type=jnp.float32,
                 precision=lax.Precision.HIGHEST))


def _mm3_body(sA_ref, sB_ref, deg_ref, b2_ref, wc_ref, bc_ref,
              oA_ref, oB_ref):
    dis = lax.rsqrt(deg_ref[...])
    z = dis * (sA_ref[...] + sB_ref[...]) + b2_ref[...]
    ab = jnp.dot(z, wc_ref[...], preferred_element_type=jnp.float32,
                 precision=lax.Precision.HIGHEST) \
        + bc_ref[...]
    oA_ref[...] = ab[:, :DH]
    oB_ref[...] = ab[:, DH:]


BR = 3200


def _red_body(p_ref, b_ref, o_ref):
    o_ref[...] = jnp.sum(p_ref[...], axis=1, keepdims=True) + b_ref[...]


def kernel(x, edge_index, W1, b1, W2, b2, lpW1, lpb1, lpW2, lpb2):
    src = edge_index[0]
    dst = edge_index[1]

    pad = E_PAD - E
    src1 = jnp.concatenate([src, jnp.zeros((pad,), jnp.int32)])
    dst1 = jnp.concatenate([dst, jnp.full((pad,), TRASH, jnp.int32)])
    src2d = src1.reshape(NROWS2D, CHUNK)
    dst2d = dst1.reshape(NROWS2D, CHUNK)
    xp = jnp.pad(x, ((0, N_PAD - N), (0, 0)))
    zerosD = jnp.zeros((N_PAD, DOUT), jnp.float32)

    deg_mat = pl.pallas_call(
        _deg_body,
        grid=(E // BE,),
        in_specs=[pl.BlockSpec((BE, 1), lambda i: (i, 0))],
        out_specs=pl.BlockSpec((NHI, 128), lambda i: (0, 0)),
        out_shape=jax.ShapeDtypeStruct((NHI, 128), jnp.float32),
    )(dst.reshape(E, 1))
    deg_col = deg_mat.reshape(N_PAD, 1)

    h1pL, h1pR = pl.pallas_call(
        _mm1_body,
        grid=(N_PAD // BM,),
        in_specs=[
            pl.BlockSpec((BM, DIN), lambda i: (i, 0)),
            pl.BlockSpec((DIN, DH), lambda i: (0, 0)),
            pl.BlockSpec((BM, 1), lambda i: (i, 0)),
        ],
        out_specs=[pl.BlockSpec((BM, DH // 2), lambda i: (i, 0))] * 2,
        out_shape=[jax.ShapeDtypeStruct((N_PAD, DH // 2), jnp.float32)] * 2,
    )(xp, W1, deg_col)

    s1L, s1R = _conv_scatter_1(h1pL, h1pR, src1, dst1)

    h2p = pl.pallas_call(
        _mm2_body,
        grid=(N_PAD // BM,),
        in_specs=[
            pl.BlockSpec((BM, DH // 2), lambda i: (i, 0)),
            pl.BlockSpec((BM, DH // 2), lambda i: (i, 0)),
            pl.BlockSpec((BM, 1), lambda i: (i, 0)),
            pl.BlockSpec((1, DH), lambda i: (0, 0)),
            pl.BlockSpec((DH, DOUT), lambda i: (0, 0)),
        ],
        out_specs=pl.BlockSpec((BM, DOUT), lambda i: (i, 0)),
        out_shape=jax.ShapeDtypeStruct((N_PAD, DOUT), jnp.float32),
    )(s1L, s1R, deg_col, b1.reshape(1, DH), W2)

    s2A, s2B = _conv_scatter_2(h2p, zerosD, src1, dst1)

    wcat = jnp.concatenate([lpW1[:DOUT], lpW1[DOUT:]], axis=1)
    bcat = jnp.concatenate([lpb1, jnp.zeros((DH,), jnp.float32)]).reshape(1, -1)

    A, B = pl.pallas_call(
        _mm3_body,
        grid=(N_PAD // BM,),
        in_specs=[
            pl.BlockSpec((BM, DOUT), lambda i: (i, 0)),
            pl.BlockSpec((BM, DOUT), lambda i: (i, 0)),
            pl.BlockSpec((BM, 1), lambda i: (i, 0)),
            pl.BlockSpec((1, DOUT), lambda i: (0, 0)),
            pl.BlockSpec((DOUT, 2 * DH), lambda i: (0, 0)),
            pl.BlockSpec((1, 2 * DH), lambda i: (0, 0)),
        ],
        out_specs=[pl.BlockSpec((BM, DH), lambda i: (i, 0))] * 2,
        out_shape=[jax.ShapeDtypeStruct((N_PAD, DH), jnp.float32)] * 2,
    )(s2A, s2B, deg_col, b2.reshape(1, DOUT), wcat, bcat)

    w16 = lpW2.reshape(16, 16)

    P = _dec_kernel(A, B, src2d, dst2d, w16)

    out = pl.pallas_call(
        _red_body,
        grid=(E // BR,),
        in_specs=[
            pl.BlockSpec((BR, 16), lambda i: (i, 0)),
            pl.BlockSpec((1, 1), lambda i: (0, 0)),
        ],
        out_specs=pl.BlockSpec((BR, 1), lambda i: (i, 0)),
        out_shape=jax.ShapeDtypeStruct((E, 1), jnp.float32),
    )(P, lpb2.reshape(1, 1))

    return out


# speedup vs baseline: 1.0305x; 1.0242x over previous
import functools

import jax
import jax.numpy as jnp
from jax import lax
from jax.experimental import pallas as pl
from jax.experimental.pallas import tpu as pltpu
from jax.experimental.pallas import tpu_sc as plsc

N = 10000
E = 320000
DIN = 128
DH = 256
DOUT = 128

NC = 2
NS = 16
CHUNK = 128

CPT1 = 160
CPT2 = 80
NROWS2D = NS * CPT1
E_PAD = NROWS2D * CHUNK

N_PAD = 10240
ROWS_TILE = N_PAD // NS
ROWS_IO = 128
IO_CHUNKS = ROWS_TILE // ROWS_IO
TRASH = N

HCH = 64

_mesh = plsc.VectorSubcoreMesh(core_axis_name="c", subcore_axis_name="s")


def _make_conv_scatter(dh, feature_split):
    cpt = CPT1 if feature_split else CPT2

    @functools.partial(
        pl.kernel,
        mesh=_mesh,
        out_type=jax.ShapeDtypeStruct((2, N_PAD, dh), jnp.float32),
        scratch_types=[
            pltpu.VMEM((CHUNK,), jnp.int32),
            pltpu.VMEM((CHUNK,), jnp.int32),
            pltpu.VMEM((CHUNK, dh), jnp.float32),
            pltpu.VMEM((ROWS_IO, dh), jnp.float32),
            pltpu.VMEM_SHARED((N_PAD, dh), jnp.float32),
            pltpu.SemaphoreType.DMA,
        ],
    )
    def conv_kernel(in_hbm, src_hbm, dst_hbm, out_hbm,
                    sidx_v, didx_v, rows_v, bounce_v, acc_sh, sem):
        c = lax.axis_index("c")
        s = lax.axis_index("s")
        if feature_split:
            gat_hbm = in_hbm.at[c]
            cbase = s * CPT1
        else:
            gat_hbm = in_hbm.at[0]
            cbase = (s * NC + c) * CPT2

        for i in range(IO_CHUNKS):
            rb = s * ROWS_TILE + i * ROWS_IO
            pltpu.sync_copy(in_hbm.at[c, pl.ds(rb, ROWS_IO)], bounce_v)
            pltpu.sync_copy(bounce_v, acc_sh.at[pl.ds(rb, ROWS_IO)])
        plsc.subcore_barrier()

        ebase = cbase * CHUNK

        def body(j, carry):
            off = ebase + j * CHUNK
            pltpu.sync_copy(src_hbm.at[pl.ds(off, CHUNK)], sidx_v)
            pltpu.sync_copy(dst_hbm.at[pl.ds(off, CHUNK)], didx_v)
            pltpu.async_copy(gat_hbm.at[sidx_v], rows_v, sem).wait()
            pltpu.sync_copy(rows_v, acc_sh.at[didx_v], add=True)
            return carry

        lax.fori_loop(0, cpt, body, 0)
        plsc.subcore_barrier()
        for i in range(IO_CHUNKS):
            rb = s * ROWS_TILE + i * ROWS_IO
            pltpu.sync_copy(acc_sh.at[pl.ds(rb, ROWS_IO)], bounce_v)
            pltpu.sync_copy(bounce_v, out_hbm.at[c, pl.ds(rb, ROWS_IO)])

    return conv_kernel


_conv_scatter_1 = _make_conv_scatter(DH // 2, True)
_conv_scatter_2 = _make_conv_scatter(DOUT, False)


N_HCH = CPT2 * 2


@functools.partial(
    pl.kernel,
    mesh=_mesh,
    out_type=jax.ShapeDtypeStruct((E_PAD, 16), jnp.float32),
    scratch_types=[
        pltpu.VMEM((CPT2, CHUNK), jnp.int32),
        pltpu.VMEM((CPT2, CHUNK), jnp.int32),
        pltpu.VMEM((HCH, 2 * DOUT), jnp.float32),
        pltpu.VMEM((HCH, 2 * DOUT), jnp.float32),
        pltpu.VMEM((HCH, 2 * DOUT), jnp.float32),
        pltpu.VMEM((HCH, 2 * DOUT), jnp.float32),
        pltpu.VMEM((16, 16), jnp.float32),
        pltpu.VMEM((HCH, 16), jnp.float32),
        pltpu.VMEM((HCH, 16), jnp.float32),
        pltpu.SemaphoreType.DMA,
        pltpu.SemaphoreType.DMA,
        pltpu.SemaphoreType.DMA,
        pltpu.SemaphoreType.DMA,
        pltpu.SemaphoreType.DMA,
        pltpu.SemaphoreType.DMA,
    ],
)
def _dec_kernel(a_hbm, b_hbm, src2d_hbm, dst2d_hbm, w_hbm, p_hbm,
                sidx_v, didx_v, a0_v, a1_v, b0_v, b1_v, w_v, p0_v, p1_v,
                asem0, asem1, bsem0, bsem1, psem0, psem1):
    c = lax.axis_index("c")
    s = lax.axis_index("s")
    wid = s * NC + c
    cbase = wid * CPT2
    ebase = wid * (CPT2 * CHUNK)

    av = (a0_v, a1_v)
    bv = (b0_v, b1_v)
    pv = (p0_v, p1_v)
    asems = (asem0, asem1)
    bsems = (bsem0, bsem1)
    psems = (psem0, psem1)

    pltpu.sync_copy(w_hbm, w_v)
    wregs = [w_v[i, :] for i in range(16)]
    zero = jnp.zeros((16,), jnp.float32)

    pltpu.sync_copy(src2d_hbm.at[pl.ds(cbase, CPT2)], sidx_v)
    pltpu.sync_copy(dst2d_hbm.at[pl.ds(cbase, CPT2)], didx_v)

    def sidx_of(j):
        return sidx_v.at[j // 2, pl.ds((j % 2) * HCH, HCH)]

    def didx_of(j):
        return didx_v.at[j // 2, pl.ds((j % 2) * HCH, HCH)]

    for k in range(2):
        pltpu.async_copy(a_hbm.at[sidx_of(k)], av[k], asems[k])
        pltpu.async_copy(b_hbm.at[didx_of(k)], bv[k], bsems[k])

    def body(i, carry):
        for k in range(2):
            j = i * 2 + k
            pltpu.make_async_copy(a_hbm.at[sidx_of(j)], av[k], asems[k]).wait()
            pltpu.make_async_copy(b_hbm.at[didx_of(j)], bv[k], bsems[k]).wait()

            @pl.when(j >= 2)
            def _():
                pltpu.make_async_copy(
                    pv[k], p_hbm.at[pl.ds(ebase + j * HCH, HCH)],
                    psems[k]).wait()

            def row(r, rc):
                acc = [zero, zero, zero, zero]
                for jj in range(16):
                    t = jnp.maximum(
                        av[k][r, pl.ds(jj * 16, 16)]
                        + bv[k][r, pl.ds(jj * 16, 16)], 0.0)
                    acc[jj % 4] = t * wregs[jj] + acc[jj % 4]
                pv[k][r, :] = (acc[0] + acc[1]) + (acc[2] + acc[3])
                return rc

            lax.fori_loop(0, HCH, row, 0)
            pltpu.async_copy(pv[k], p_hbm.at[pl.ds(ebase + j * HCH, HCH)],
                             psems[k])
            jn = j + 2

            @pl.when(jn < N_HCH)
            def _():
                pltpu.async_copy(a_hbm.at[sidx_of(jn)], av[k], asems[k])
                pltpu.async_copy(b_hbm.at[didx_of(jn)], bv[k], bsems[k])
        return carry

    lax.fori_loop(0, N_HCH // 2, body, 0)
    for k in range(2):
        pltpu.make_async_copy(
            pv[k], p_hbm.at[pl.ds(ebase + (N_HCH - 2 + k) * HCH, HCH)],
            psems[k]).wait()


BM = 2048
BE = 2000
NHI = N_PAD // 128


def _deg_body(dst_ref, o_ref):
    i = pl.program_id(0)

    @pl.when(i == 0)
    def _():
        o_ref[...] = jnp.full((NHI, 128), 1.0, jnp.float32)

    d = dst_ref[...]
    hi = lax.broadcasted_iota(jnp.int32, (BE, NHI), 1)
    lo = lax.broadcasted_iota(jnp.int32, (BE, 128), 1)
    u = (d // 128 == hi).astype(jnp.float32)
    v = (d % 128 == lo).astype(jnp.float32)
    o_ref[...] += lax.dot_general(u, v, (((0,), (0,)), ((), ())),
                                  preferred_element_type=jnp.float32)


def _mm1_body(x_ref, w_ref, deg_ref, o_ref):
    dis = lax.rsqrt(deg_ref[...])
    h = jnp.dot(x_ref[...], w_ref[...], preferred_element_type=jnp.float32)
    o_ref[0] = dis * h[:, : DH // 2]
    o_ref[1] = dis * h[:, DH // 2:]


def _mm2_body(s1_ref, deg_ref, b1_ref, w2_ref, o_ref):
    dis = lax.rsqrt(deg_ref[...])
    b1 = b1_ref[...]
    w2 = w2_ref[...]
    zL = jnp.maximum(dis * s1_ref[0] + b1[:, : DH // 2], 0.0)
    zR = jnp.maximum(dis * s1_ref[1] + b1[:, DH // 2:], 0.0)
    o_ref[0] = dis * (
        jnp.dot(zL, w2[: DH // 2], preferred_element_type=jnp.float32)
        + jnp.dot(zR, w2[DH // 2:], preferred_element_type=jnp.float32))
    o_ref[1] = jnp.zeros((BM, DOUT), jnp.float32)


def _mm3_body(s2_ref, deg_ref, b2_ref, wc_ref, bc_ref,
              oA_ref, oB_ref):
    dis = lax.rsqrt(deg_ref[...])
    z = dis * (s2_ref[0] + s2_ref[1]) + b2_ref[...]
    ab = jnp.dot(z, wc_ref[...], preferred_element_type=jnp.float32) \
        + bc_ref[...]
    oA_ref[...] = ab[:, :DH]
    oB_ref[...] = ab[:, DH:]


BR = 3200


def _red_body(p_ref, b_ref, o_ref):
    o_ref[...] = jnp.sum(p_ref[...], axis=1, keepdims=True) + b_ref[...]


def kernel(x, edge_index, W1, b1, W2, b2, lpW1, lpb1, lpW2, lpb2):
    src = edge_index[0]
    dst = edge_index[1]

    pad = E_PAD - E
    src1 = jnp.concatenate([src, jnp.zeros((pad,), jnp.int32)])
    dst1 = jnp.concatenate([dst, jnp.full((pad,), TRASH, jnp.int32)])
    src2d = src1.reshape(NROWS2D, CHUNK)
    dst2d = dst1.reshape(NROWS2D, CHUNK)
    xp = jnp.pad(x, ((0, N_PAD - N), (0, 0)))

    deg_mat = pl.pallas_call(
        _deg_body,
        grid=(E // BE,),
        in_specs=[pl.BlockSpec((BE, 1), lambda i: (i, 0))],
        out_specs=pl.BlockSpec((NHI, 128), lambda i: (0, 0)),
        out_shape=jax.ShapeDtypeStruct((NHI, 128), jnp.float32),
    )(dst.reshape(E, 1))
    deg_col = deg_mat.reshape(N_PAD, 1)

    h1p = pl.pallas_call(
        _mm1_body,
        grid=(N_PAD // BM,),
        in_specs=[
            pl.BlockSpec((BM, DIN), lambda i: (i, 0)),
            pl.BlockSpec((DIN, DH), lambda i: (0, 0)),
            pl.BlockSpec((BM, 1), lambda i: (i, 0)),
        ],
        out_specs=pl.BlockSpec((2, BM, DH // 2), lambda i: (0, i, 0)),
        out_shape=jax.ShapeDtypeStruct((2, N_PAD, DH // 2), jnp.float32),
    )(xp, W1, deg_col)

    s1 = _conv_scatter_1(h1p, src1, dst1)

    h2p = pl.pallas_call(
        _mm2_body,
        grid=(N_PAD // BM,),
        in_specs=[
            pl.BlockSpec((2, BM, DH // 2), lambda i: (0, i, 0)),
            pl.BlockSpec((BM, 1), lambda i: (i, 0)),
            pl.BlockSpec((1, DH), lambda i: (0, 0)),
            pl.BlockSpec((DH, DOUT), lambda i: (0, 0)),
        ],
        out_specs=pl.BlockSpec((2, BM, DOUT), lambda i: (0, i, 0)),
        out_shape=jax.ShapeDtypeStruct((2, N_PAD, DOUT), jnp.float32),
    )(s1, deg_col, b1.reshape(1, DH), W2)

    s2 = _conv_scatter_2(h2p, src1, dst1)

    wcat = jnp.concatenate([lpW1[:DOUT], lpW1[DOUT:]], axis=1)
    bcat = jnp.concatenate([lpb1, jnp.zeros((DH,), jnp.float32)]).reshape(1, -1)

    A, B = pl.pallas_call(
        _mm3_body,
        grid=(N_PAD // BM,),
        in_specs=[
            pl.BlockSpec((2, BM, DOUT), lambda i: (0, i, 0)),
            pl.BlockSpec((BM, 1), lambda i: (i, 0)),
            pl.BlockSpec((1, DOUT), lambda i: (0, 0)),
            pl.BlockSpec((DOUT, 2 * DH), lambda i: (0, 0)),
            pl.BlockSpec((1, 2 * DH), lambda i: (0, 0)),
        ],
        out_specs=[pl.BlockSpec((BM, DH), lambda i: (i, 0))] * 2,
        out_shape=[jax.ShapeDtypeStruct((N_PAD, DH), jnp.float32)] * 2,
    )(s2, deg_col, b2.reshape(1, DOUT), wcat, bcat)

    w16 = lpW2.reshape(16, 16)

    P = _dec_kernel(A, B, src2d, dst2d, w16)

    out = pl.pallas_call(
        _red_body,
        grid=(E // BR,),
        in_specs=[
            pl.BlockSpec((BR, 16), lambda i: (i, 0)),
            pl.BlockSpec((1, 1), lambda i: (0, 0)),
        ],
        out_specs=pl.BlockSpec((BR, 1), lambda i: (i, 0)),
        out_shape=jax.ShapeDtypeStruct((E, 1), jnp.float32),
    )(P, lpb2.reshape(1, 1))

    return out
